# Initial kernel scaffold; baseline (speedup 1.0000x reference)
import jax
import jax.numpy as jnp
from jax.experimental import pallas as pl


def kernel(user_id, item_id, context_id, mlp_user, mlp_item, gmf_user, gmf_item, W1, b1, W2, b2, W3, b3, Wout, bout):
    raise NotImplementedError("write your pallas kernel here")


# speedup vs baseline: 1.9825x; 1.9825x over previous
import jax
import jax.numpy as jnp
from jax import lax
from jax.experimental import pallas as pl
from jax.experimental.pallas import tpu as pltpu
from jax.experimental.pallas import tpu_sc as plsc

NC = 2
NS = 16
NW = NC * NS
CH = 128
N_PARTS = 11


def _sc_gather(idx_w, mlp_user, mlp_item, gmf_user, gmf_item, nch):
    D = mlp_user.shape[1]

    def body(idx_hbm, mu_hbm, mi_hbm, gu_hbm, gi_hbm, out_hbm, idx_v, buf_v, sem):
        wid = lax.axis_index("s") * NC + lax.axis_index("c")
        pltpu.sync_copy(idx_hbm.at[wid], idx_v)

        def gather_part(tbl, g):
            cps = [
                pltpu.async_copy(tbl.at[idx_v.at[g, ci]], buf_v.at[ci], sem)
                for ci in range(nch)
            ]
            for cp in cps:
                cp.wait()
            pltpu.sync_copy(buf_v, out_hbm.at[g, wid])

        def mu_body(g, carry):
            gather_part(mu_hbm, g)
            return carry

        lax.fori_loop(0, 8, mu_body, 0)
        gather_part(mi_hbm, 8)
        gather_part(gu_hbm, 9)
        gather_part(gi_hbm, 10)

    fn = pl.kernel(
        body,
        out_type=jax.ShapeDtypeStruct((N_PARTS, NW, nch, CH, D), jnp.float32),
        mesh=plsc.VectorSubcoreMesh(
            core_axis_name="c", subcore_axis_name="s",
            num_cores=NC, num_subcores=NS,
        ),
        scratch_types=[
            pltpu.VMEM((N_PARTS, nch, CH), jnp.int32),
            pltpu.VMEM((nch, CH, D), jnp.float32),
            pltpu.SemaphoreType.DMA,
        ],
        compiler_params=pltpu.CompilerParams(use_tc_tiling_on_sc=False),
    )
    return fn(idx_w, mlp_user, mlp_item, gmf_user, gmf_item)


def _mlp_body(p_ref, w1_ref, b1_ref, w2_ref, b2_ref, w3_ref, b3_ref,
              wo_ref, bo_ref, o_ref):
    acc = jnp.dot(p_ref[0], w1_ref[0], preferred_element_type=jnp.float32)
    for j in range(1, 9):
        acc = acc + jnp.dot(p_ref[j], w1_ref[j],
                            preferred_element_type=jnp.float32)
    h1 = jnp.maximum(acc + b1_ref[...], 0.0)
    h2 = jnp.maximum(
        jnp.dot(h1, w2_ref[...], preferred_element_type=jnp.float32)
        + b2_ref[...], 0.0)
    h3 = jnp.maximum(
        jnp.dot(h2, w3_ref[...], preferred_element_type=jnp.float32)
        + b3_ref[...], 0.0)
    gmf = p_ref[9] * p_ref[10]
    wo = wo_ref[...]
    out = (jnp.sum(gmf * wo[:, :64], axis=1)
           + jnp.sum(h3 * wo[:, 64:], axis=1))
    o_ref[...] = out + bo_ref[0]


def _mlp(parts, w1r, b1, w2, b2, w3, b3, wo, bo, blk):
    B = parts.shape[1]
    return pl.pallas_call(
        _mlp_body,
        grid=(B // blk,),
        in_specs=[
            pl.BlockSpec((N_PARTS, blk, 64), lambda i: (0, i, 0)),
            pl.BlockSpec((9, 64, 256), lambda i: (0, 0, 0)),
            pl.BlockSpec((1, 256), lambda i: (0, 0)),
            pl.BlockSpec((256, 128), lambda i: (0, 0)),
            pl.BlockSpec((1, 128), lambda i: (0, 0)),
            pl.BlockSpec((128, 64), lambda i: (0, 0)),
            pl.BlockSpec((1, 64), lambda i: (0, 0)),
            pl.BlockSpec((1, 128), lambda i: (0, 0)),
            pl.BlockSpec(memory_space=pltpu.SMEM),
        ],
        out_specs=pl.BlockSpec((blk,), lambda i: (i,)),
        out_shape=jax.ShapeDtypeStruct((B,), jnp.float32),
        compiler_params=pltpu.CompilerParams(
            dimension_semantics=("arbitrary",)),
    )(parts, w1r, b1, w2, b2, w3, b3, wo, bo)


def kernel(user_id, item_id, context_id, mlp_user, mlp_item, gmf_user,
           gmf_item, W1, b1, W2, b2, W3, b3, Wout, bout):
    B = user_id.shape[0]
    user_id = user_id.astype(jnp.int32)
    item_id = item_id.astype(jnp.int32)
    ctx_t = context_id.astype(jnp.int32).T
    idx_all = jnp.concatenate(
        [user_id[None], ctx_t, item_id[None], user_id[None], item_id[None]],
        axis=0)
    nch = B // NW // CH
    idx_w = idx_all.reshape(N_PARTS, NW, nch, CH).transpose(1, 0, 2, 3)

    parts5 = _sc_gather(idx_w, mlp_user, mlp_item, gmf_user, gmf_item, nch)
    parts = parts5.reshape(N_PARTS, B, 64)

    w1r = W1.reshape(9, 64, 256)[jnp.array([0, 2, 3, 4, 5, 6, 7, 8, 1])]
    return _mlp(parts, w1r, b1.reshape(1, 256), W2, b2.reshape(1, 128),
                W3, b3.reshape(1, 64), Wout.reshape(1, 128), bout, blk=512)


# speedup vs baseline: 2.6098x; 1.3164x over previous
import jax
import jax.numpy as jnp
from jax import lax
from jax.experimental import pallas as pl
from jax.experimental.pallas import tpu as pltpu
from jax.experimental.pallas import tpu_sc as plsc

NC = 2
NS = 16
NW = NC * NS
CH = 128
N_IDX = 9
N_PAIR = 6


def _sc_gather(idx_w, user_tab, item_tab, nch):
    def body(idx_hbm, ut_hbm, it_hbm, out_hbm, idx_v, buf_v, sem):
        wid = lax.axis_index("s") * NC + lax.axis_index("c")
        pltpu.sync_copy(idx_hbm.at[wid], idx_v)

        jobs = [
            (0, ut_hbm, [(0, 0, 128)]),
            (8, it_hbm, [(1, 0, 128)]),
            (1, ut_hbm, [(2, 0, 64)]),
            (2, ut_hbm, [(2, 64, 64)]),
            (3, ut_hbm, [(3, 0, 64)]),
            (4, ut_hbm, [(3, 64, 64)]),
            (5, ut_hbm, [(4, 0, 64)]),
            (6, ut_hbm, [(4, 64, 64)]),
            (7, ut_hbm, [(5, 0, 64), (5, 64, 64)]),
        ]

        def do_chunk(ci, carry):
            for wave in (jobs[:5], jobs[5:]):
                cps = [
                    pltpu.async_copy(tbl.at[idx_v.at[r, ci]], buf_v.at[bi], sem)
                    for bi, (r, tbl, _) in enumerate(wave)
                ]
                for cp in cps:
                    cp.wait()
                for bi, (_, _, writes) in enumerate(wave):
                    for (p, off, w) in writes:
                        src = buf_v.at[bi] if w == 128 else \
                            buf_v.at[bi, :, pl.ds(0, 64)]
                        pltpu.sync_copy(
                            src, out_hbm.at[p, wid, ci, :, pl.ds(off, w)])
            return carry

        lax.fori_loop(0, nch, do_chunk, 0)

    fn = pl.kernel(
        body,
        out_type=jax.ShapeDtypeStruct((N_PAIR, NW, nch, CH, 128), jnp.float32),
        mesh=plsc.VectorSubcoreMesh(
            core_axis_name="c", subcore_axis_name="s",
            num_cores=NC, num_subcores=NS,
        ),
        scratch_types=[
            pltpu.VMEM((N_IDX, nch, CH), jnp.int32),
            pltpu.VMEM((5, CH, 128), jnp.float32),
            pltpu.SemaphoreType.DMA,
        ],
        compiler_params=pltpu.CompilerParams(use_tc_tiling_on_sc=False),
    )
    return fn(idx_w, user_tab, item_tab)


def _mlp_body(p_ref, w1_ref, b1_ref, w2_ref, b2_ref, w3_ref, b3_ref,
              wo_ref, bo_ref, o_ref):
    acc = jnp.dot(p_ref[0], w1_ref[0], preferred_element_type=jnp.float32)
    for j in range(1, N_PAIR):
        acc = acc + jnp.dot(p_ref[j], w1_ref[j],
                            preferred_element_type=jnp.float32)
    h1 = jnp.maximum(acc + b1_ref[...], 0.0)
    h2 = jnp.maximum(
        jnp.dot(h1, w2_ref[...], preferred_element_type=jnp.float32)
        + b2_ref[...], 0.0)
    h3 = jnp.maximum(
        jnp.dot(h2, w3_ref[...], preferred_element_type=jnp.float32)
        + b3_ref[...], 0.0)
    gmf = p_ref[0][:, 64:] * p_ref[1][:, 64:]
    wo = wo_ref[...]
    out = (jnp.sum(gmf * wo[:, :64], axis=1)
           + jnp.sum(h3 * wo[:, 64:], axis=1))
    o_ref[...] = out + bo_ref[0]


def _mlp(pairs, w1p, b1, w2, b2, w3, b3, wo, bo, blk):
    B = pairs.shape[1]
    return pl.pallas_call(
        _mlp_body,
        grid=(B // blk,),
        in_specs=[
            pl.BlockSpec((N_PAIR, blk, 128), lambda i: (0, i, 0)),
            pl.BlockSpec((N_PAIR, 128, 256), lambda i: (0, 0, 0)),
            pl.BlockSpec((1, 256), lambda i: (0, 0)),
            pl.BlockSpec((256, 128), lambda i: (0, 0)),
            pl.BlockSpec((1, 128), lambda i: (0, 0)),
            pl.BlockSpec((128, 64), lambda i: (0, 0)),
            pl.BlockSpec((1, 64), lambda i: (0, 0)),
            pl.BlockSpec((1, 128), lambda i: (0, 0)),
            pl.BlockSpec(memory_space=pltpu.SMEM),
        ],
        out_specs=pl.BlockSpec((blk,), lambda i: (i,)),
        out_shape=jax.ShapeDtypeStruct((B,), jnp.float32),
        compiler_params=pltpu.CompilerParams(
            dimension_semantics=("arbitrary",)),
    )(pairs, w1p, b1, w2, b2, w3, b3, wo, bo)


def kernel(user_id, item_id, context_id, mlp_user, mlp_item, gmf_user,
           gmf_item, W1, b1, W2, b2, W3, b3, Wout, bout):
    B = user_id.shape[0]
    user_id = user_id.astype(jnp.int32)
    item_id = item_id.astype(jnp.int32)
    ctx_t = context_id.astype(jnp.int32).T

    user_tab = jnp.concatenate([mlp_user, gmf_user], axis=1)
    item_tab = jnp.concatenate([mlp_item, gmf_item], axis=1)

    idx_all = jnp.concatenate([user_id[None], ctx_t, item_id[None]], axis=0)
    nch = B // NW // CH
    idx_w = idx_all.reshape(N_IDX, NW, nch, CH).transpose(1, 0, 2, 3)

    pairs5 = _sc_gather(idx_w, user_tab, item_tab, nch)
    pairs = pairs5.reshape(N_PAIR, B, 128)

    blocks = W1.reshape(9, 64, 256)
    z = jnp.zeros((64, 256), W1.dtype)
    w1p = jnp.stack([
        jnp.concatenate([blocks[0], z]),
        jnp.concatenate([blocks[1], z]),
        jnp.concatenate([blocks[2], blocks[3]]),
        jnp.concatenate([blocks[4], blocks[5]]),
        jnp.concatenate([blocks[6], blocks[7]]),
        jnp.concatenate([blocks[8], z]),
    ])
    return _mlp(pairs, w1p, b1.reshape(1, 256), W2, b2.reshape(1, 128),
                W3, b3.reshape(1, 64), Wout.reshape(1, 128), bout, blk=512)
